# baseline (speedup 1.0000x reference)
import functools

import jax
import jax.numpy as jnp
from jax import lax
from jax._src import config as _jax_config
from jax.experimental import pallas as pl
from jax.experimental.pallas import tpu as pltpu
from jax.experimental.pallas import tpu_sc as plsc

N = 10000; E = 320000; D = 128; H = 128; OUT = 64; ENC = 100; K = 128
CC = 0.25; EPS = 1e-5
NC, NS = 2, 16
NW = NC * NS
CHUNK = 64
NG = 5
SUB = 32
EPW = NG * SUB * CHUNK
EPAD = NW * EPW
RPS = 632
NP = RPS * NS
PADROW = NP - 8


def _gelu(x):
    return 0.5 * x * (1.0 + lax.erf(x * (2.0 ** -0.5)))


def _dyt(x, a, g, b):
    return g * jnp.tanh(a * x) + b


def _tc1_body(x_ref, g_ref, b_ref, w1_ref, b1_ref, w2_ref, b2_ref,
              da_ref, dg_ref, db_ref, h_ref):
    x = x_ref[...]
    mu = jnp.mean(x, axis=0, keepdims=True)
    var = jnp.mean((x - mu) ** 2, axis=0, keepdims=True)
    xn = (x - mu) / jnp.sqrt(var + EPS) * g_ref[...] + b_ref[...]
    t = _gelu(jnp.dot(xn, w1_ref[...]) + b1_ref[...])
    t = _gelu(jnp.dot(t, w2_ref[...]) + b2_ref[...])
    h_ref[...] = _dyt(t, da_ref[0, 0], dg_ref[...], db_ref[...])


def _tc1(x, p):
    return pl.pallas_call(
        _tc1_body,
        out_shape=jax.ShapeDtypeStruct((N, H), jnp.float32),
    )(x, p['bn_g'].reshape(1, D), p['bn_b'].reshape(1, D),
      p['ffin_w1'], p['ffin_b1'].reshape(1, 2 * H),
      p['ffin_w2'], p['ffin_b2'].reshape(1, H),
      p['ffin_dyt_a'].reshape(1, 1), p['ffin_dyt_g'].reshape(1, H),
      p['ffin_dyt_b'].reshape(1, H))


def _tc2_body(h_ref, s0_ref, s1_ref, c0_ref, c1_ref,
              wl_ref, bl_ref, wr_ref, ga_ref, gg_ref, gb_ref, o_ref):
    cnt = jnp.maximum(c0_ref[...] + c1_ref[...], 1.0)
    agg = (s0_ref[...] + s1_ref[...]) / cnt
    y = jnp.dot(agg, wl_ref[...]) + bl_ref[...] + jnp.dot(h_ref[...], wr_ref[...])
    y = _gelu(y)
    mean = jnp.mean(y, axis=0, keepdims=True)
    out = y - ga_ref[...] * mean
    v = jnp.mean(out ** 2, axis=0, keepdims=True)
    o_ref[...] = out / jnp.sqrt(v + EPS) * gg_ref[...] + gb_ref[...]


def _tc2(h, s0, s1, c0, c1, wl, bl, wr, ga, gg, gb):
    return pl.pallas_call(
        _tc2_body,
        out_shape=jax.ShapeDtypeStruct((N, H), jnp.float32),
    )(h, s0, s1, c0, c1, wl, bl.reshape(1, H), wr,
      ga.reshape(1, H), gg.reshape(1, H), gb.reshape(1, H))


def _tc3_body(x1_ref, s0_ref, s1_ref, c0_ref, c1_ref, xaa_ref,
              wl_ref, bl_ref, wr_ref, ga_ref, gg_ref, gb_ref,
              d0a_ref, d0g_ref, d0b_ref, lw1_ref, lb1_ref, lw2_ref, lb2_ref,
              d1a_ref, d1g_ref, d1b_ref,
              ow1_ref, ob1_ref, ow2_ref, ob2_ref, ow3_ref, ob3_ref,
              oda_ref, odg_ref, odb_ref, cb_ref, zq_ref, loss_ref):
    x1 = x1_ref[...]
    cnt = jnp.maximum(c0_ref[...] + c1_ref[...], 1.0)
    agg = (s0_ref[...] + s1_ref[...]) / cnt
    y = jnp.dot(agg, wl_ref[...]) + bl_ref[...] + jnp.dot(x1, wr_ref[...])
    y = _gelu(y)
    mean = jnp.mean(y, axis=0, keepdims=True)
    out = y - ga_ref[...] * mean
    v = jnp.mean(out ** 2, axis=0, keepdims=True)
    x2 = out / jnp.sqrt(v + EPS) * gg_ref[...] + gb_ref[...]

    cat = jnp.concatenate([x1, x2], axis=1)
    t = _dyt(cat, d0a_ref[0, 0], d0g_ref[...], d0b_ref[...])
    t = _gelu(jnp.dot(t, lw1_ref[...]) + lb1_ref[...])
    t = _gelu(jnp.dot(t, lw2_ref[...]) + lb2_ref[...])
    t = _dyt(t, d1a_ref[0, 0], d1g_ref[...], d1b_ref[...])

    u = jnp.concatenate([t, xaa_ref[...]], axis=1)
    u = _gelu(jnp.dot(u, ow1_ref[...]) + ob1_ref[...])
    u = _gelu(jnp.dot(u, ow2_ref[...]) + ob2_ref[...])
    u = _gelu(jnp.dot(u, ow3_ref[...]) + ob3_ref[...])
    x = _dyt(u, oda_ref[0, 0], odg_ref[...], odb_ref[...])

    cb = cb_ref[...]
    d = (jnp.sum(x ** 2, axis=1, keepdims=True)
         + jnp.sum(cb ** 2, axis=1)[None, :]
         - 2.0 * jnp.dot(x, cb.T))
    md = jnp.min(d, axis=1, keepdims=True)
    ii = lax.broadcasted_iota(jnp.int32, (N, K), 1)
    idx = jnp.min(jnp.where(d == md, ii, K), axis=1, keepdims=True)
    onehot = (ii == idx).astype(jnp.float32)
    q = jnp.dot(onehot, cb)
    loss = (1.0 + CC) * jnp.mean((q - x) ** 2)
    zq_ref[...] = q
    loss_ref[...] = jnp.broadcast_to(loss, (1, 1))


def _tc3(x1, s0, s1, c0, c1, xaa, p):
    return pl.pallas_call(
        _tc3_body,
        out_shape=(jax.ShapeDtypeStruct((N, OUT), jnp.float32),
                   jax.ShapeDtypeStruct((1, 1), jnp.float32)),
    )(x1, s0, s1, c0, c1, xaa,
      p['c2_wl'], p['c2_bl'].reshape(1, H), p['c2_wr'],
      p['gn2_a'].reshape(1, H), p['gn2_g'].reshape(1, H), p['gn2_b'].reshape(1, H),
      p['lin_dyt0_a'].reshape(1, 1), p['lin_dyt0_g'].reshape(1, 2 * H),
      p['lin_dyt0_b'].reshape(1, 2 * H),
      p['lin_w1'], p['lin_b1'].reshape(1, ENC), p['lin_w2'], p['lin_b2'].reshape(1, ENC),
      p['lin_dyt1_a'].reshape(1, 1), p['lin_dyt1_g'].reshape(1, ENC),
      p['lin_dyt1_b'].reshape(1, ENC),
      p['od_w1'], p['od_b1'].reshape(1, ENC), p['od_w2'], p['od_b2'].reshape(1, ENC // 2),
      p['od_w3'], p['od_b3'].reshape(1, OUT),
      p['od_dyt_a'].reshape(1, 1), p['od_dyt_g'].reshape(1, OUT),
      p['od_dyt_b'].reshape(1, OUT), p['codebook'])


def _make_sc_segsum(with_cnt):
    mesh = plsc.VectorSubcoreMesh(core_axis_name="c", subcore_axis_name="s")
    out_type = [jax.ShapeDtypeStruct((NC, NP, H), jnp.float32)]
    scratch = [
        pltpu.VMEM((SUB, CHUNK), jnp.int32),
        pltpu.VMEM((SUB, CHUNK), jnp.int32),
        pltpu.VMEM((CHUNK, H), jnp.float32),
        pltpu.VMEM((CHUNK, H), jnp.float32),
        pltpu.VMEM_SHARED((NP, H), jnp.float32),
        pltpu.SemaphoreType.DMA,
        pltpu.SemaphoreType.DMA,
        pltpu.SemaphoreType.DMA,
        pltpu.SemaphoreType.DMA,
    ]
    if with_cnt:
        out_type.append(jax.ShapeDtypeStruct((NC, NP, 16), jnp.float32))
        scratch += [pltpu.VMEM((CHUNK, 16), jnp.float32),
                    pltpu.VMEM_SHARED((NP, 16), jnp.float32)]

    @functools.partial(
        pl.kernel, mesh=mesh,
        compiler_params=pltpu.CompilerParams(use_tc_tiling_on_sc=False),
        out_type=tuple(out_type) if with_cnt else out_type[0],
        scratch_types=scratch,
    )
    def k(h_hbm, src_hbm, dst_hbm, zacc_hbm, *rest):
        if with_cnt:
            (zcnt_hbm, ones_hbm, acc_out, cnt_out, srcv, dstv, rows0, rows1,
             acc_sh, g0, g1, s0, s1, onesv, cnt_sh) = rest
        else:
            (acc_out, srcv, dstv, rows0, rows1, acc_sh, g0, g1, s0, s1) = rest
        cid = lax.axis_index("c").astype(jnp.int32)
        sid = lax.axis_index("s").astype(jnp.int32)
        wid = cid * jnp.int32(NS) + sid
        row0 = sid * jnp.int32(RPS)
        pltpu.sync_copy(zacc_hbm.at[pl.ds(row0, RPS)],
                        acc_sh.at[pl.ds(row0, RPS)])
        if with_cnt:
            pltpu.sync_copy(zcnt_hbm.at[pl.ds(row0, RPS)],
                            cnt_sh.at[pl.ds(row0, RPS)])
            pltpu.sync_copy(ones_hbm, onesv)
        plsc.subcore_barrier()

        bufs = (rows0, rows1)
        sg = (g0, g1)
        ss = (s0, s1)

        @pl.loop(0, NG)
        def _(g):
            pltpu.sync_copy(src_hbm.at[wid, g], srcv)
            pltpu.sync_copy(dst_hbm.at[wid, g], dstv)
            pltpu.async_copy(h_hbm.at[srcv.at[0]], rows0, g0)

            @pl.loop(0, SUB, step=2)
            def _(j):
                for t in range(2):
                    jj = j + jnp.int32(t)

                    def _proc():
                        pltpu.make_async_copy(h_hbm.at[srcv.at[jj]],
                                              bufs[t], sg[t]).wait()

                        @pl.when(jj + 1 < SUB)
                        def _():
                            @pl.when(jj >= 1)
                            def _():
                                pltpu.make_async_copy(
                                    bufs[1 - t], acc_sh.at[dstv.at[jj - 1]],
                                    ss[1 - t]).wait()

                            pltpu.async_copy(h_hbm.at[srcv.at[jj + 1]],
                                             bufs[1 - t], sg[1 - t])

                        pltpu.async_copy(bufs[t], acc_sh.at[dstv.at[jj]],
                                         ss[t], add=True)
                        if with_cnt:
                            pltpu.sync_copy(onesv, cnt_sh.at[dstv.at[jj]],
                                            add=True)

                    if t == 0:
                        _proc()
                    else:
                        pl.when(jj < SUB)(_proc)

            pltpu.make_async_copy(bufs[1], acc_sh.at[dstv.at[SUB - 2]],
                                  ss[1]).wait()
            pltpu.make_async_copy(bufs[0], acc_sh.at[dstv.at[SUB - 1]],
                                  ss[0]).wait()

        plsc.subcore_barrier()
        pltpu.sync_copy(acc_sh.at[pl.ds(row0, RPS)],
                        acc_out.at[cid].at[pl.ds(row0, RPS)])
        if with_cnt:
            pltpu.sync_copy(cnt_sh.at[pl.ds(row0, RPS)],
                            cnt_out.at[cid].at[pl.ds(row0, RPS)])

    return k


_sc_segsum_cnt = _make_sc_segsum(True)
_sc_segsum_nocnt = _make_sc_segsum(False)


def kernel(x_res, x_aa, params, edge_index):
    with _jax_config.enable_x64(False):
        return _kernel_impl(x_res, x_aa, params, edge_index)


def _kernel_impl(x_res, x_aa, params, edge_index):
    p = params
    npad = EPAD - E
    src3 = jnp.concatenate(
        [edge_index[0].astype(jnp.int32), jnp.zeros((npad,), jnp.int32)]
    ).reshape(NW, NG, SUB, CHUNK)
    dst3 = jnp.concatenate(
        [edge_index[1].astype(jnp.int32),
         jnp.full((npad,), PADROW, jnp.int32)]
    ).reshape(NW, NG, SUB, CHUNK)
    zacc = jnp.zeros((NP, H), jnp.float32)
    zcnt = jnp.zeros((NP, 16), jnp.float32)
    ones = jnp.ones((CHUNK, 16), jnp.float32)

    h0 = _tc1(x_res.astype(jnp.float32), p)
    s_a, c_a = _sc_segsum_cnt(h0, src3, dst3, zacc, zcnt, ones)
    c0 = c_a[0, :N, :1]
    c1 = c_a[1, :N, :1]
    x1 = _tc2(h0, s_a[0, :N], s_a[1, :N], c0, c1,
              p['c1_wl'], p['c1_bl'], p['c1_wr'],
              p['gn1_a'], p['gn1_g'], p['gn1_b'])
    s_b = _sc_segsum_nocnt(x1, src3, dst3, zacc)
    zq, loss = _tc3(x1, s_b[0, :N], s_b[1, :N], c0, c1, x_aa.astype(jnp.float32), p)
    return zq, loss.reshape(())


# speedup vs baseline: 1.0018x; 1.0018x over previous
import functools

import jax
import jax.numpy as jnp
from jax import lax
from jax._src import config as _jax_config
from jax.experimental import pallas as pl
from jax.experimental.pallas import tpu as pltpu
from jax.experimental.pallas import tpu_sc as plsc

N = 10000; E = 320000; D = 128; H = 128; OUT = 64; ENC = 100; K = 128
CC = 0.25; EPS = 1e-5
NC, NS = 2, 16
NW = NC * NS
CHUNK = 64
NG = 5
SUB = 32
EPW = NG * SUB * CHUNK
EPAD = NW * EPW
RPS = 632
NP = RPS * NS
PADROW = NP - 8


def _gelu(x):
    return 0.5 * x * (1.0 + lax.erf(x * (2.0 ** -0.5)))


def _dyt(x, a, g, b):
    return g * jnp.tanh(a * x) + b


def _tc1_body(x_ref, g_ref, b_ref, w1_ref, b1_ref, w2_ref, b2_ref,
              da_ref, dg_ref, db_ref, h_ref):
    x = x_ref[...]
    mu = jnp.mean(x, axis=0, keepdims=True)
    var = jnp.mean((x - mu) ** 2, axis=0, keepdims=True)
    xn = (x - mu) / jnp.sqrt(var + EPS) * g_ref[...] + b_ref[...]
    t = _gelu(jnp.dot(xn, w1_ref[...]) + b1_ref[...])
    t = _gelu(jnp.dot(t, w2_ref[...]) + b2_ref[...])
    h_ref[...] = _dyt(t, da_ref[0, 0], dg_ref[...], db_ref[...])


def _tc1(x, p):
    return pl.pallas_call(
        _tc1_body,
        out_shape=jax.ShapeDtypeStruct((N, H), jnp.float32),
    )(x, p['bn_g'].reshape(1, D), p['bn_b'].reshape(1, D),
      p['ffin_w1'], p['ffin_b1'].reshape(1, 2 * H),
      p['ffin_w2'], p['ffin_b2'].reshape(1, H),
      p['ffin_dyt_a'].reshape(1, 1), p['ffin_dyt_g'].reshape(1, H),
      p['ffin_dyt_b'].reshape(1, H))


def _tc2_body(h_ref, s0_ref, s1_ref, c0_ref, c1_ref,
              wl_ref, bl_ref, wr_ref, ga_ref, gg_ref, gb_ref, o_ref):
    cnt = jnp.maximum(c0_ref[...] + c1_ref[...], 1.0)
    agg = (s0_ref[...] + s1_ref[...]) / cnt
    y = jnp.dot(agg, wl_ref[...]) + bl_ref[...] + jnp.dot(h_ref[...], wr_ref[...])
    y = _gelu(y)
    mean = jnp.mean(y, axis=0, keepdims=True)
    out = y - ga_ref[...] * mean
    v = jnp.mean(out ** 2, axis=0, keepdims=True)
    o_ref[...] = out / jnp.sqrt(v + EPS) * gg_ref[...] + gb_ref[...]


def _tc2(h, s0, s1, c0, c1, wl, bl, wr, ga, gg, gb):
    return pl.pallas_call(
        _tc2_body,
        out_shape=jax.ShapeDtypeStruct((N, H), jnp.float32),
    )(h, s0, s1, c0, c1, wl, bl.reshape(1, H), wr,
      ga.reshape(1, H), gg.reshape(1, H), gb.reshape(1, H))


def _tc3_body(x1_ref, s0_ref, s1_ref, c0_ref, c1_ref, xaa_ref,
              wl_ref, bl_ref, wr_ref, ga_ref, gg_ref, gb_ref,
              d0a_ref, d0g_ref, d0b_ref, lw1_ref, lb1_ref, lw2_ref, lb2_ref,
              d1a_ref, d1g_ref, d1b_ref,
              ow1_ref, ob1_ref, ow2_ref, ob2_ref, ow3_ref, ob3_ref,
              oda_ref, odg_ref, odb_ref, cb_ref, zq_ref, loss_ref):
    x1 = x1_ref[...]
    cnt = jnp.maximum(c0_ref[...] + c1_ref[...], 1.0)
    agg = (s0_ref[...] + s1_ref[...]) / cnt
    y = jnp.dot(agg, wl_ref[...]) + bl_ref[...] + jnp.dot(x1, wr_ref[...])
    y = _gelu(y)
    mean = jnp.mean(y, axis=0, keepdims=True)
    out = y - ga_ref[...] * mean
    v = jnp.mean(out ** 2, axis=0, keepdims=True)
    x2 = out / jnp.sqrt(v + EPS) * gg_ref[...] + gb_ref[...]

    cat = jnp.concatenate([x1, x2], axis=1)
    t = _dyt(cat, d0a_ref[0, 0], d0g_ref[...], d0b_ref[...])
    t = _gelu(jnp.dot(t, lw1_ref[...]) + lb1_ref[...])
    t = _gelu(jnp.dot(t, lw2_ref[...]) + lb2_ref[...])
    t = _dyt(t, d1a_ref[0, 0], d1g_ref[...], d1b_ref[...])

    u = jnp.concatenate([t, xaa_ref[...]], axis=1)
    u = _gelu(jnp.dot(u, ow1_ref[...]) + ob1_ref[...])
    u = _gelu(jnp.dot(u, ow2_ref[...]) + ob2_ref[...])
    u = _gelu(jnp.dot(u, ow3_ref[...]) + ob3_ref[...])
    x = _dyt(u, oda_ref[0, 0], odg_ref[...], odb_ref[...])

    cb = cb_ref[...]
    d = (jnp.sum(x ** 2, axis=1, keepdims=True)
         + jnp.sum(cb ** 2, axis=1)[None, :]
         - 2.0 * jnp.dot(x, cb.T))
    md = jnp.min(d, axis=1, keepdims=True)
    ii = lax.broadcasted_iota(jnp.int32, (N, K), 1)
    idx = jnp.min(jnp.where(d == md, ii, K), axis=1, keepdims=True)
    onehot = (ii == idx).astype(jnp.float32)
    q = jnp.dot(onehot, cb)
    loss = (1.0 + CC) * jnp.mean((q - x) ** 2)
    zq_ref[...] = q
    loss_ref[...] = jnp.broadcast_to(loss, (1, 1))


def _tc3(x1, s0, s1, c0, c1, xaa, p):
    return pl.pallas_call(
        _tc3_body,
        out_shape=(jax.ShapeDtypeStruct((N, OUT), jnp.float32),
                   jax.ShapeDtypeStruct((1, 1), jnp.float32)),
    )(x1, s0, s1, c0, c1, xaa,
      p['c2_wl'], p['c2_bl'].reshape(1, H), p['c2_wr'],
      p['gn2_a'].reshape(1, H), p['gn2_g'].reshape(1, H), p['gn2_b'].reshape(1, H),
      p['lin_dyt0_a'].reshape(1, 1), p['lin_dyt0_g'].reshape(1, 2 * H),
      p['lin_dyt0_b'].reshape(1, 2 * H),
      p['lin_w1'], p['lin_b1'].reshape(1, ENC), p['lin_w2'], p['lin_b2'].reshape(1, ENC),
      p['lin_dyt1_a'].reshape(1, 1), p['lin_dyt1_g'].reshape(1, ENC),
      p['lin_dyt1_b'].reshape(1, ENC),
      p['od_w1'], p['od_b1'].reshape(1, ENC), p['od_w2'], p['od_b2'].reshape(1, ENC // 2),
      p['od_w3'], p['od_b3'].reshape(1, OUT),
      p['od_dyt_a'].reshape(1, 1), p['od_dyt_g'].reshape(1, OUT),
      p['od_dyt_b'].reshape(1, OUT), p['codebook'])


def _make_sc_segsum(with_cnt):
    mesh = plsc.VectorSubcoreMesh(core_axis_name="c", subcore_axis_name="s")
    out_type = [jax.ShapeDtypeStruct((NC, NP, H), jnp.float32)]
    scratch = [
        pltpu.VMEM((SUB, CHUNK), jnp.int32),
        pltpu.VMEM((SUB, CHUNK), jnp.int32),
        pltpu.VMEM((CHUNK, H), jnp.float32),
        pltpu.VMEM((CHUNK, H), jnp.float32),
        pltpu.VMEM_SHARED((NP, H), jnp.float32),
        pltpu.SemaphoreType.DMA,
        pltpu.SemaphoreType.DMA,
        pltpu.SemaphoreType.DMA,
        pltpu.SemaphoreType.DMA,
    ]
    if with_cnt:
        out_type.append(jax.ShapeDtypeStruct((NC, NP, 16), jnp.float32))
        scratch += [pltpu.VMEM((CHUNK, 16), jnp.float32),
                    pltpu.VMEM_SHARED((NP, 16), jnp.float32)]

    @functools.partial(
        pl.kernel, mesh=mesh,
        compiler_params=pltpu.CompilerParams(use_tc_tiling_on_sc=False),
        out_type=tuple(out_type) if with_cnt else out_type[0],
        scratch_types=scratch,
    )
    def k(h_hbm, src_hbm, dst_hbm, zacc_hbm, *rest):
        if with_cnt:
            (zcnt_hbm, ones_hbm, acc_out, cnt_out, srcv, dstv, rows0, rows1,
             acc_sh, g0, g1, s0, s1, onesv, cnt_sh) = rest
        else:
            (acc_out, srcv, dstv, rows0, rows1, acc_sh, g0, g1, s0, s1) = rest
        cid = lax.axis_index("c").astype(jnp.int32)
        sid = lax.axis_index("s").astype(jnp.int32)
        wid = cid * jnp.int32(NS) + sid
        row0 = sid * jnp.int32(RPS)
        pltpu.sync_copy(zacc_hbm.at[pl.ds(row0, RPS)],
                        acc_sh.at[pl.ds(row0, RPS)])
        if with_cnt:
            pltpu.sync_copy(zcnt_hbm.at[pl.ds(row0, RPS)],
                            cnt_sh.at[pl.ds(row0, RPS)])
            pltpu.sync_copy(ones_hbm, onesv)
        plsc.subcore_barrier()

        bufs = (rows0, rows1)
        sg = (g0, g1)
        ss = (s0, s1)

        @pl.loop(0, NG)
        def _(g):
            pltpu.sync_copy(src_hbm.at[wid, g], srcv)
            pltpu.sync_copy(dst_hbm.at[wid, g], dstv)
            pltpu.async_copy(h_hbm.at[srcv.at[0]], rows0, g0)

            @pl.loop(0, SUB, step=2)
            def _(j):
                for t in range(2):
                    jj = j + jnp.int32(t)

                    def _proc():
                        pltpu.make_async_copy(h_hbm.at[srcv.at[jj]],
                                              bufs[t], sg[t]).wait()

                        @pl.when(jj + 1 < SUB)
                        def _():
                            @pl.when(jj >= 1)
                            def _():
                                pltpu.make_async_copy(
                                    bufs[1 - t], acc_sh.at[dstv.at[jj - 1]],
                                    ss[1 - t]).wait()

                            pltpu.async_copy(h_hbm.at[srcv.at[jj + 1]],
                                             bufs[1 - t], sg[1 - t])

                        pltpu.async_copy(bufs[t], acc_sh.at[dstv.at[jj]],
                                         ss[t], add=True)
                        if with_cnt:
                            pltpu.sync_copy(onesv, cnt_sh.at[dstv.at[jj]],
                                            add=True)

                    if t == 0:
                        _proc()
                    else:
                        pl.when(jj < SUB)(_proc)

            pltpu.make_async_copy(bufs[1], acc_sh.at[dstv.at[SUB - 2]],
                                  ss[1]).wait()
            pltpu.make_async_copy(bufs[0], acc_sh.at[dstv.at[SUB - 1]],
                                  ss[0]).wait()

        plsc.subcore_barrier()
        pltpu.sync_copy(acc_sh.at[pl.ds(row0, RPS)],
                        acc_out.at[cid].at[pl.ds(row0, RPS)])
        if with_cnt:
            pltpu.sync_copy(cnt_sh.at[pl.ds(row0, RPS)],
                            cnt_out.at[cid].at[pl.ds(row0, RPS)])

    return k


_sc_segsum_cnt = _make_sc_segsum(True)
_sc_segsum_nocnt = _make_sc_segsum(False)


def kernel(x_res, x_aa, params, edge_index):
    with _jax_config.enable_x64(False):
        return _kernel_impl(x_res, x_aa, params, edge_index)


def _kernel_impl(x_res, x_aa, params, edge_index):
    p = params
    npad = EPAD - E
    src3 = jnp.concatenate(
        [edge_index[0].astype(jnp.int32), jnp.zeros((npad,), jnp.int32)]
    ).reshape(NW, NG, SUB, CHUNK)
    dst3 = jnp.concatenate(
        [edge_index[1].astype(jnp.int32),
         N + (jnp.arange(npad, dtype=jnp.int32) % (NP - N))]
    ).reshape(NW, NG, SUB, CHUNK)
    zacc = jnp.zeros((NP, H), jnp.float32)
    zcnt = jnp.zeros((NP, 16), jnp.float32)
    ones = jnp.ones((CHUNK, 16), jnp.float32)

    h0 = _tc1(x_res.astype(jnp.float32), p)
    s_a, c_a = _sc_segsum_cnt(h0, src3, dst3, zacc, zcnt, ones)
    c0 = c_a[0, :N, :1]
    c1 = c_a[1, :N, :1]
    x1 = _tc2(h0, s_a[0, :N], s_a[1, :N], c0, c1,
              p['c1_wl'], p['c1_bl'], p['c1_wr'],
              p['gn1_a'], p['gn1_g'], p['gn1_b'])
    s_b = _sc_segsum_nocnt(x1, src3, dst3, zacc)
    zq, loss = _tc3(x1, s_b[0, :N], s_b[1, :N], c0, c1, x_aa.astype(jnp.float32), p)
    return zq, loss.reshape(())


# speedup vs baseline: 2.2834x; 2.2794x over previous
import functools

import jax
import jax.numpy as jnp
from jax import lax
from jax._src import config as _jax_config
from jax.experimental import pallas as pl
from jax.experimental.pallas import tpu as pltpu
from jax.experimental.pallas import tpu_sc as plsc

N = 10000; E = 320000; D = 128; H = 128; OUT = 64; ENC = 100; K = 128
CC = 0.25; EPS = 1e-5
NC, NS = 2, 16
NW = NC * NS
CHUNK = 64
NG = 5
SUB = 32
EPW = NG * SUB * CHUNK
EPAD = NW * EPW
RPS = 632
NP = RPS * NS
PADROW = NP - 8


def _gelu(x):
    return 0.5 * x * (1.0 + lax.erf(x * (2.0 ** -0.5)))


def _dyt(x, a, g, b):
    return g * jnp.tanh(a * x) + b


def _tc1_body(x_ref, g_ref, b_ref, w1_ref, b1_ref, w2_ref, b2_ref,
              da_ref, dg_ref, db_ref, h_ref):
    x = x_ref[...]
    mu = jnp.mean(x, axis=0, keepdims=True)
    var = jnp.mean((x - mu) ** 2, axis=0, keepdims=True)
    xn = (x - mu) / jnp.sqrt(var + EPS) * g_ref[...] + b_ref[...]
    t = _gelu(jnp.dot(xn, w1_ref[...]) + b1_ref[...])
    t = _gelu(jnp.dot(t, w2_ref[...]) + b2_ref[...])
    h_ref[...] = _dyt(t, da_ref[0, 0], dg_ref[...], db_ref[...])


def _tc1(x, p):
    return pl.pallas_call(
        _tc1_body,
        out_shape=jax.ShapeDtypeStruct((N, H), jnp.float32),
    )(x, p['bn_g'].reshape(1, D), p['bn_b'].reshape(1, D),
      p['ffin_w1'], p['ffin_b1'].reshape(1, 2 * H),
      p['ffin_w2'], p['ffin_b2'].reshape(1, H),
      p['ffin_dyt_a'].reshape(1, 1), p['ffin_dyt_g'].reshape(1, H),
      p['ffin_dyt_b'].reshape(1, H))


def _tc2_body(h_ref, s0_ref, s1_ref, c0_ref, c1_ref,
              wl_ref, bl_ref, wr_ref, ga_ref, gg_ref, gb_ref, o_ref):
    cnt = jnp.maximum(c0_ref[...] + c1_ref[...], 1.0)
    agg = (s0_ref[...] + s1_ref[...]) / cnt
    y = jnp.dot(agg, wl_ref[...]) + bl_ref[...] + jnp.dot(h_ref[...], wr_ref[...])
    y = _gelu(y)
    mean = jnp.mean(y, axis=0, keepdims=True)
    out = y - ga_ref[...] * mean
    v = jnp.mean(out ** 2, axis=0, keepdims=True)
    o_ref[...] = out / jnp.sqrt(v + EPS) * gg_ref[...] + gb_ref[...]


def _tc2(h, s0, s1, c0, c1, wl, bl, wr, ga, gg, gb):
    return pl.pallas_call(
        _tc2_body,
        out_shape=jax.ShapeDtypeStruct((N, H), jnp.float32),
    )(h, s0, s1, c0, c1, wl, bl.reshape(1, H), wr,
      ga.reshape(1, H), gg.reshape(1, H), gb.reshape(1, H))


def _tc3_body(x1_ref, s0_ref, s1_ref, c0_ref, c1_ref, xaa_ref,
              wl_ref, bl_ref, wr_ref, ga_ref, gg_ref, gb_ref,
              d0a_ref, d0g_ref, d0b_ref, lw1_ref, lb1_ref, lw2_ref, lb2_ref,
              d1a_ref, d1g_ref, d1b_ref,
              ow1_ref, ob1_ref, ow2_ref, ob2_ref, ow3_ref, ob3_ref,
              oda_ref, odg_ref, odb_ref, cb_ref, zq_ref, loss_ref):
    x1 = x1_ref[...]
    cnt = jnp.maximum(c0_ref[...] + c1_ref[...], 1.0)
    agg = (s0_ref[...] + s1_ref[...]) / cnt
    y = jnp.dot(agg, wl_ref[...]) + bl_ref[...] + jnp.dot(x1, wr_ref[...])
    y = _gelu(y)
    mean = jnp.mean(y, axis=0, keepdims=True)
    out = y - ga_ref[...] * mean
    v = jnp.mean(out ** 2, axis=0, keepdims=True)
    x2 = out / jnp.sqrt(v + EPS) * gg_ref[...] + gb_ref[...]

    cat = jnp.concatenate([x1, x2], axis=1)
    t = _dyt(cat, d0a_ref[0, 0], d0g_ref[...], d0b_ref[...])
    t = _gelu(jnp.dot(t, lw1_ref[...]) + lb1_ref[...])
    t = _gelu(jnp.dot(t, lw2_ref[...]) + lb2_ref[...])
    t = _dyt(t, d1a_ref[0, 0], d1g_ref[...], d1b_ref[...])

    u = jnp.concatenate([t, xaa_ref[...]], axis=1)
    u = _gelu(jnp.dot(u, ow1_ref[...]) + ob1_ref[...])
    u = _gelu(jnp.dot(u, ow2_ref[...]) + ob2_ref[...])
    u = _gelu(jnp.dot(u, ow3_ref[...]) + ob3_ref[...])
    x = _dyt(u, oda_ref[0, 0], odg_ref[...], odb_ref[...])

    cb = cb_ref[...]
    d = (jnp.sum(x ** 2, axis=1, keepdims=True)
         + jnp.sum(cb ** 2, axis=1)[None, :]
         - 2.0 * jnp.dot(x, cb.T))
    md = jnp.min(d, axis=1, keepdims=True)
    ii = lax.broadcasted_iota(jnp.int32, (N, K), 1)
    idx = jnp.min(jnp.where(d == md, ii, K), axis=1, keepdims=True)
    onehot = (ii == idx).astype(jnp.float32)
    q = jnp.dot(onehot, cb)
    loss = (1.0 + CC) * jnp.mean((q - x) ** 2)
    zq_ref[...] = q
    loss_ref[...] = jnp.broadcast_to(loss, (1, 1))


def _tc3(x1, s0, s1, c0, c1, xaa, p):
    return pl.pallas_call(
        _tc3_body,
        out_shape=(jax.ShapeDtypeStruct((N, OUT), jnp.float32),
                   jax.ShapeDtypeStruct((1, 1), jnp.float32)),
    )(x1, s0, s1, c0, c1, xaa,
      p['c2_wl'], p['c2_bl'].reshape(1, H), p['c2_wr'],
      p['gn2_a'].reshape(1, H), p['gn2_g'].reshape(1, H), p['gn2_b'].reshape(1, H),
      p['lin_dyt0_a'].reshape(1, 1), p['lin_dyt0_g'].reshape(1, 2 * H),
      p['lin_dyt0_b'].reshape(1, 2 * H),
      p['lin_w1'], p['lin_b1'].reshape(1, ENC), p['lin_w2'], p['lin_b2'].reshape(1, ENC),
      p['lin_dyt1_a'].reshape(1, 1), p['lin_dyt1_g'].reshape(1, ENC),
      p['lin_dyt1_b'].reshape(1, ENC),
      p['od_w1'], p['od_b1'].reshape(1, ENC), p['od_w2'], p['od_b2'].reshape(1, ENC // 2),
      p['od_w3'], p['od_b3'].reshape(1, OUT),
      p['od_dyt_a'].reshape(1, 1), p['od_dyt_g'].reshape(1, OUT),
      p['od_dyt_b'].reshape(1, OUT), p['codebook'])


def _make_sc_segsum(with_cnt):
    mesh = plsc.VectorSubcoreMesh(core_axis_name="c", subcore_axis_name="s")
    out_type = [jax.ShapeDtypeStruct((NC, NP, H), jnp.float32)]
    scratch = [
        pltpu.VMEM((SUB, CHUNK), jnp.int32),
        pltpu.VMEM((SUB, CHUNK), jnp.int32),
        pltpu.VMEM((CHUNK, H), jnp.float32),
        pltpu.VMEM((CHUNK, H), jnp.float32),
        pltpu.VMEM_SHARED((NP, H), jnp.float32),
        pltpu.SemaphoreType.DMA,
        pltpu.SemaphoreType.DMA,
        pltpu.SemaphoreType.DMA,
        pltpu.SemaphoreType.DMA,
    ]
    if with_cnt:
        out_type.append(jax.ShapeDtypeStruct((NC, NP, 16), jnp.float32))
        scratch += [pltpu.VMEM((CHUNK, 16), jnp.float32),
                    pltpu.VMEM_SHARED((NP, 16), jnp.float32)]

    @functools.partial(
        pl.kernel, mesh=mesh,
        compiler_params=pltpu.CompilerParams(use_tc_tiling_on_sc=False),
        out_type=tuple(out_type) if with_cnt else out_type[0],
        scratch_types=scratch,
    )
    def k(h_hbm, src_hbm, dst_hbm, zacc_hbm, *rest):
        if with_cnt:
            (zcnt_hbm, ones_hbm, acc_out, cnt_out, srcv, dstv, rows0, rows1,
             acc_sh, g0, g1, s0, s1, onesv, cnt_sh) = rest
        else:
            (acc_out, srcv, dstv, rows0, rows1, acc_sh, g0, g1, s0, s1) = rest
        cid = lax.axis_index("c").astype(jnp.int32)
        sid = lax.axis_index("s").astype(jnp.int32)
        wid = cid * jnp.int32(NS) + sid
        row0 = sid * jnp.int32(RPS)
        pltpu.sync_copy(zacc_hbm.at[pl.ds(row0, RPS)],
                        acc_sh.at[pl.ds(row0, RPS)])
        if with_cnt:
            pltpu.sync_copy(zcnt_hbm.at[pl.ds(row0, RPS)],
                            cnt_sh.at[pl.ds(row0, RPS)])
            pltpu.sync_copy(ones_hbm, onesv)
        plsc.subcore_barrier()

        bufs = (rows0, rows1)
        sg = (g0, g1)
        ss = (s0, s1)

        @pl.loop(0, NG)
        def _(g):
            pltpu.sync_copy(src_hbm.at[wid, g], srcv)
            pltpu.sync_copy(dst_hbm.at[wid, g], dstv)
            pltpu.async_copy(h_hbm.at[srcv.at[0]], rows0, g0)

            @pl.loop(0, SUB, step=2)
            def _(j):
                for t in range(2):
                    jj = j + jnp.int32(t)

                    def _proc():
                        pltpu.make_async_copy(h_hbm.at[srcv.at[jj]],
                                              bufs[t], sg[t]).wait()

                        @pl.when(jj + 1 < SUB)
                        def _():
                            @pl.when(jj >= 1)
                            def _():
                                pltpu.make_async_copy(
                                    bufs[1 - t], acc_sh.at[dstv.at[jj - 1]],
                                    ss[1 - t]).wait()

                            pltpu.async_copy(h_hbm.at[srcv.at[jj + 1]],
                                             bufs[1 - t], sg[1 - t])

                        pltpu.async_copy(bufs[t], acc_sh.at[dstv.at[jj]],
                                         ss[t], add=True)
                        if with_cnt:
                            pltpu.sync_copy(onesv, cnt_sh.at[dstv.at[jj]],
                                            add=True)

                    if t == 0:
                        _proc()
                    else:
                        pl.when(jj < SUB)(_proc)

            pltpu.make_async_copy(bufs[1], acc_sh.at[dstv.at[SUB - 2]],
                                  ss[1]).wait()
            pltpu.make_async_copy(bufs[0], acc_sh.at[dstv.at[SUB - 1]],
                                  ss[0]).wait()

        plsc.subcore_barrier()
        pltpu.sync_copy(acc_sh.at[pl.ds(row0, RPS)],
                        acc_out.at[cid].at[pl.ds(row0, RPS)])
        if with_cnt:
            pltpu.sync_copy(cnt_sh.at[pl.ds(row0, RPS)],
                            cnt_out.at[cid].at[pl.ds(row0, RPS)])

    return k


_sc_segsum_cnt = _make_sc_segsum(True)
_sc_segsum_nocnt = _make_sc_segsum(False)


def kernel(x_res, x_aa, params, edge_index):
    with _jax_config.enable_x64(False):
        return _kernel_impl(x_res, x_aa, params, edge_index)


def _kernel_impl(x_res, x_aa, params, edge_index):
    p = params
    npad = EPAD - E
    src3 = jnp.concatenate(
        [edge_index[0].astype(jnp.int32),
         jnp.arange(npad, dtype=jnp.int32) % N]
    ).reshape(NW, NG, SUB, CHUNK)
    dst3 = jnp.concatenate(
        [edge_index[1].astype(jnp.int32),
         N + (jnp.arange(npad, dtype=jnp.int32) % (NP - N))]
    ).reshape(NW, NG, SUB, CHUNK)
    zacc = jnp.zeros((NP, H), jnp.float32)
    zcnt = jnp.zeros((NP, 16), jnp.float32)
    ones = jnp.ones((CHUNK, 16), jnp.float32)

    h0 = _tc1(x_res.astype(jnp.float32), p)
    s_a, c_a = _sc_segsum_cnt(h0, src3, dst3, zacc, zcnt, ones)
    c0 = c_a[0, :N, :1]
    c1 = c_a[1, :N, :1]
    x1 = _tc2(h0, s_a[0, :N], s_a[1, :N], c0, c1,
              p['c1_wl'], p['c1_bl'], p['c1_wr'],
              p['gn1_a'], p['gn1_g'], p['gn1_b'])
    s_b = _sc_segsum_nocnt(x1, src3, dst3, zacc)
    zq, loss = _tc3(x1, s_b[0, :N], s_b[1, :N], c0, c1, x_aa.astype(jnp.float32), p)
    return zq, loss.reshape(())


# speedup vs baseline: 2.7878x; 1.2209x over previous
import functools

import jax
import jax.numpy as jnp
from jax import lax
from jax._src import config as _jax_config
from jax.experimental import pallas as pl
from jax.experimental.pallas import tpu as pltpu
from jax.experimental.pallas import tpu_sc as plsc

N = 10000; E = 320000; D = 128; H = 128; OUT = 64; ENC = 100; K = 128
CC = 0.25; EPS = 1e-5
NC, NS = 2, 16
NW = NC * NS
CHUNK = 128
NG = 5
SUB = 16
EPW = NG * SUB * CHUNK
EPAD = NW * EPW
RPS = 632
NP = RPS * NS
PADROW = NP - 8


def _gelu(x):
    return 0.5 * x * (1.0 + lax.erf(x * (2.0 ** -0.5)))


def _dyt(x, a, g, b):
    return g * jnp.tanh(a * x) + b


def _tc1_body(x_ref, g_ref, b_ref, w1_ref, b1_ref, w2_ref, b2_ref,
              da_ref, dg_ref, db_ref, h_ref):
    x = x_ref[...]
    mu = jnp.mean(x, axis=0, keepdims=True)
    var = jnp.mean((x - mu) ** 2, axis=0, keepdims=True)
    xn = (x - mu) / jnp.sqrt(var + EPS) * g_ref[...] + b_ref[...]
    t = _gelu(jnp.dot(xn, w1_ref[...]) + b1_ref[...])
    t = _gelu(jnp.dot(t, w2_ref[...]) + b2_ref[...])
    h_ref[...] = _dyt(t, da_ref[0, 0], dg_ref[...], db_ref[...])


def _tc1(x, p):
    return pl.pallas_call(
        _tc1_body,
        out_shape=jax.ShapeDtypeStruct((N, H), jnp.float32),
    )(x, p['bn_g'].reshape(1, D), p['bn_b'].reshape(1, D),
      p['ffin_w1'], p['ffin_b1'].reshape(1, 2 * H),
      p['ffin_w2'], p['ffin_b2'].reshape(1, H),
      p['ffin_dyt_a'].reshape(1, 1), p['ffin_dyt_g'].reshape(1, H),
      p['ffin_dyt_b'].reshape(1, H))


def _tc2_body(h_ref, s0_ref, s1_ref, c0_ref, c1_ref,
              wl_ref, bl_ref, wr_ref, ga_ref, gg_ref, gb_ref, o_ref):
    cnt = jnp.maximum(c0_ref[...] + c1_ref[...], 1.0)
    agg = (s0_ref[...] + s1_ref[...]) / cnt
    y = jnp.dot(agg, wl_ref[...]) + bl_ref[...] + jnp.dot(h_ref[...], wr_ref[...])
    y = _gelu(y)
    mean = jnp.mean(y, axis=0, keepdims=True)
    out = y - ga_ref[...] * mean
    v = jnp.mean(out ** 2, axis=0, keepdims=True)
    o_ref[...] = out / jnp.sqrt(v + EPS) * gg_ref[...] + gb_ref[...]


def _tc2(h, s0, s1, c0, c1, wl, bl, wr, ga, gg, gb):
    return pl.pallas_call(
        _tc2_body,
        out_shape=jax.ShapeDtypeStruct((N, H), jnp.float32),
    )(h, s0, s1, c0, c1, wl, bl.reshape(1, H), wr,
      ga.reshape(1, H), gg.reshape(1, H), gb.reshape(1, H))


def _tc3_body(x1_ref, s0_ref, s1_ref, c0_ref, c1_ref, xaa_ref,
              wl_ref, bl_ref, wr_ref, ga_ref, gg_ref, gb_ref,
              d0a_ref, d0g_ref, d0b_ref, lw1_ref, lb1_ref, lw2_ref, lb2_ref,
              d1a_ref, d1g_ref, d1b_ref,
              ow1_ref, ob1_ref, ow2_ref, ob2_ref, ow3_ref, ob3_ref,
              oda_ref, odg_ref, odb_ref, cb_ref, zq_ref, loss_ref):
    x1 = x1_ref[...]
    cnt = jnp.maximum(c0_ref[...] + c1_ref[...], 1.0)
    agg = (s0_ref[...] + s1_ref[...]) / cnt
    y = jnp.dot(agg, wl_ref[...]) + bl_ref[...] + jnp.dot(x1, wr_ref[...])
    y = _gelu(y)
    mean = jnp.mean(y, axis=0, keepdims=True)
    out = y - ga_ref[...] * mean
    v = jnp.mean(out ** 2, axis=0, keepdims=True)
    x2 = out / jnp.sqrt(v + EPS) * gg_ref[...] + gb_ref[...]

    cat = jnp.concatenate([x1, x2], axis=1)
    t = _dyt(cat, d0a_ref[0, 0], d0g_ref[...], d0b_ref[...])
    t = _gelu(jnp.dot(t, lw1_ref[...]) + lb1_ref[...])
    t = _gelu(jnp.dot(t, lw2_ref[...]) + lb2_ref[...])
    t = _dyt(t, d1a_ref[0, 0], d1g_ref[...], d1b_ref[...])

    u = jnp.concatenate([t, xaa_ref[...]], axis=1)
    u = _gelu(jnp.dot(u, ow1_ref[...]) + ob1_ref[...])
    u = _gelu(jnp.dot(u, ow2_ref[...]) + ob2_ref[...])
    u = _gelu(jnp.dot(u, ow3_ref[...]) + ob3_ref[...])
    x = _dyt(u, oda_ref[0, 0], odg_ref[...], odb_ref[...])

    cb = cb_ref[...]
    d = (jnp.sum(x ** 2, axis=1, keepdims=True)
         + jnp.sum(cb ** 2, axis=1)[None, :]
         - 2.0 * jnp.dot(x, cb.T))
    md = jnp.min(d, axis=1, keepdims=True)
    ii = lax.broadcasted_iota(jnp.int32, (N, K), 1)
    idx = jnp.min(jnp.where(d == md, ii, K), axis=1, keepdims=True)
    onehot = (ii == idx).astype(jnp.float32)
    q = jnp.dot(onehot, cb)
    loss = (1.0 + CC) * jnp.mean((q - x) ** 2)
    zq_ref[...] = q
    loss_ref[...] = jnp.broadcast_to(loss, (1, 1))


def _tc3(x1, s0, s1, c0, c1, xaa, p):
    return pl.pallas_call(
        _tc3_body,
        out_shape=(jax.ShapeDtypeStruct((N, OUT), jnp.float32),
                   jax.ShapeDtypeStruct((1, 1), jnp.float32)),
    )(x1, s0, s1, c0, c1, xaa,
      p['c2_wl'], p['c2_bl'].reshape(1, H), p['c2_wr'],
      p['gn2_a'].reshape(1, H), p['gn2_g'].reshape(1, H), p['gn2_b'].reshape(1, H),
      p['lin_dyt0_a'].reshape(1, 1), p['lin_dyt0_g'].reshape(1, 2 * H),
      p['lin_dyt0_b'].reshape(1, 2 * H),
      p['lin_w1'], p['lin_b1'].reshape(1, ENC), p['lin_w2'], p['lin_b2'].reshape(1, ENC),
      p['lin_dyt1_a'].reshape(1, 1), p['lin_dyt1_g'].reshape(1, ENC),
      p['lin_dyt1_b'].reshape(1, ENC),
      p['od_w1'], p['od_b1'].reshape(1, ENC), p['od_w2'], p['od_b2'].reshape(1, ENC // 2),
      p['od_w3'], p['od_b3'].reshape(1, OUT),
      p['od_dyt_a'].reshape(1, 1), p['od_dyt_g'].reshape(1, OUT),
      p['od_dyt_b'].reshape(1, OUT), p['codebook'])


def _make_sc_segsum(with_cnt):
    mesh = plsc.VectorSubcoreMesh(core_axis_name="c", subcore_axis_name="s")
    out_type = [jax.ShapeDtypeStruct((NC, NP, H), jnp.float32)]
    scratch = [
        pltpu.VMEM((SUB, CHUNK), jnp.int32),
        pltpu.VMEM((SUB, CHUNK), jnp.int32),
        pltpu.VMEM((CHUNK, H), jnp.float32),
        pltpu.VMEM((CHUNK, H), jnp.float32),
        pltpu.VMEM_SHARED((NP, H), jnp.float32),
        pltpu.SemaphoreType.DMA,
        pltpu.SemaphoreType.DMA,
        pltpu.SemaphoreType.DMA,
        pltpu.SemaphoreType.DMA,
    ]
    if with_cnt:
        out_type.append(jax.ShapeDtypeStruct((NC, NP, 16), jnp.float32))
        scratch += [pltpu.VMEM((CHUNK, 16), jnp.float32),
                    pltpu.VMEM_SHARED((NP, 16), jnp.float32)]

    @functools.partial(
        pl.kernel, mesh=mesh,
        compiler_params=pltpu.CompilerParams(use_tc_tiling_on_sc=False),
        out_type=tuple(out_type) if with_cnt else out_type[0],
        scratch_types=scratch,
    )
    def k(h_hbm, src_hbm, dst_hbm, zacc_hbm, *rest):
        if with_cnt:
            (zcnt_hbm, ones_hbm, acc_out, cnt_out, srcv, dstv, rows0, rows1,
             acc_sh, g0, g1, s0, s1, onesv, cnt_sh) = rest
        else:
            (acc_out, srcv, dstv, rows0, rows1, acc_sh, g0, g1, s0, s1) = rest
        cid = lax.axis_index("c").astype(jnp.int32)
        sid = lax.axis_index("s").astype(jnp.int32)
        wid = cid * jnp.int32(NS) + sid
        row0 = sid * jnp.int32(RPS)
        pltpu.sync_copy(zacc_hbm.at[pl.ds(row0, RPS)],
                        acc_sh.at[pl.ds(row0, RPS)])
        if with_cnt:
            pltpu.sync_copy(zcnt_hbm.at[pl.ds(row0, RPS)],
                            cnt_sh.at[pl.ds(row0, RPS)])
            pltpu.sync_copy(ones_hbm, onesv)
        plsc.subcore_barrier()

        bufs = (rows0, rows1)
        sg = (g0, g1)
        ss = (s0, s1)

        @pl.loop(0, NG)
        def _(g):
            pltpu.sync_copy(src_hbm.at[wid, g], srcv)
            pltpu.sync_copy(dst_hbm.at[wid, g], dstv)
            pltpu.async_copy(h_hbm.at[srcv.at[0]], rows0, g0)

            @pl.loop(0, SUB, step=2)
            def _(j):
                for t in range(2):
                    jj = j + jnp.int32(t)

                    def _proc():
                        pltpu.make_async_copy(h_hbm.at[srcv.at[jj]],
                                              bufs[t], sg[t]).wait()

                        @pl.when(jj + 1 < SUB)
                        def _():
                            @pl.when(jj >= 1)
                            def _():
                                pltpu.make_async_copy(
                                    bufs[1 - t], acc_sh.at[dstv.at[jj - 1]],
                                    ss[1 - t]).wait()

                            pltpu.async_copy(h_hbm.at[srcv.at[jj + 1]],
                                             bufs[1 - t], sg[1 - t])

                        pltpu.async_copy(bufs[t], acc_sh.at[dstv.at[jj]],
                                         ss[t], add=True)
                        if with_cnt:
                            pltpu.sync_copy(onesv, cnt_sh.at[dstv.at[jj]],
                                            add=True)

                    if t == 0:
                        _proc()
                    else:
                        pl.when(jj < SUB)(_proc)

            pltpu.make_async_copy(bufs[1], acc_sh.at[dstv.at[SUB - 2]],
                                  ss[1]).wait()
            pltpu.make_async_copy(bufs[0], acc_sh.at[dstv.at[SUB - 1]],
                                  ss[0]).wait()

        plsc.subcore_barrier()
        pltpu.sync_copy(acc_sh.at[pl.ds(row0, RPS)],
                        acc_out.at[cid].at[pl.ds(row0, RPS)])
        if with_cnt:
            pltpu.sync_copy(cnt_sh.at[pl.ds(row0, RPS)],
                            cnt_out.at[cid].at[pl.ds(row0, RPS)])

    return k


_sc_segsum_cnt = _make_sc_segsum(True)
_sc_segsum_nocnt = _make_sc_segsum(False)


def kernel(x_res, x_aa, params, edge_index):
    with _jax_config.enable_x64(False):
        return _kernel_impl(x_res, x_aa, params, edge_index)


def _kernel_impl(x_res, x_aa, params, edge_index):
    p = params
    npad = EPAD - E
    src3 = jnp.concatenate(
        [edge_index[0].astype(jnp.int32),
         jnp.arange(npad, dtype=jnp.int32) % N]
    ).reshape(NW, NG, SUB, CHUNK)
    dst3 = jnp.concatenate(
        [edge_index[1].astype(jnp.int32),
         N + (jnp.arange(npad, dtype=jnp.int32) % (NP - N))]
    ).reshape(NW, NG, SUB, CHUNK)
    zacc = jnp.zeros((NP, H), jnp.float32)
    zcnt = jnp.zeros((NP, 16), jnp.float32)
    ones = jnp.ones((CHUNK, 16), jnp.float32)

    h0 = _tc1(x_res.astype(jnp.float32), p)
    s_a, c_a = _sc_segsum_cnt(h0, src3, dst3, zacc, zcnt, ones)
    c0 = c_a[0, :N, :1]
    c1 = c_a[1, :N, :1]
    x1 = _tc2(h0, s_a[0, :N], s_a[1, :N], c0, c1,
              p['c1_wl'], p['c1_bl'], p['c1_wr'],
              p['gn1_a'], p['gn1_g'], p['gn1_b'])
    s_b = _sc_segsum_nocnt(x1, src3, dst3, zacc)
    zq, loss = _tc3(x1, s_b[0, :N], s_b[1, :N], c0, c1, x_aa.astype(jnp.float32), p)
    return zq, loss.reshape(())
